# baseline (speedup 1.0000x reference)
import functools

import jax
import jax.numpy as jnp
from jax import lax
from jax.experimental import pallas as pl
from jax.experimental.pallas import tpu as pltpu
from jax.experimental.pallas import tpu_sc as plsc

VOCAB = 100000
SEQ = 2048
DIM = 128
EPS = 1e-12
BATCH = 4

NC = 2
NS = 16
NW = NC * NS
N_TOK = BATCH * SEQ
ROWS_PER_W = N_TOK // NW
CHUNK = 128
N_CHUNK = ROWS_PER_W // CHUNK
W_PER_B = SEQ // ROWS_PER_W
LANES = 16
DLANES = DIM // LANES


def _xlane_sum(x):
    idx = lax.iota(jnp.int32, LANES)
    dnums = lax.GatherDimensionNumbers(
        offset_dims=(), collapsed_slice_dims=(0,), start_index_map=(0,))
    for k in (1, 2, 4, 8):
        perm = lax.bitwise_xor(idx, jnp.int32(k))
        x = x + lax.gather(
            x, perm[:, None], dimension_numbers=dnums, slice_sizes=(1,),
            mode=lax.GatherScatterMode.PROMISE_IN_BOUNDS)
    return x


def _rsqrt(x):
    yi = jnp.int32(0x5F3759DF) - lax.shift_right_logical(
        lax.bitcast_convert_type(x, jnp.int32), 1)
    y = lax.bitcast_convert_type(yi, jnp.float32)
    for _ in range(3):
        y = y * (1.5 - 0.5 * x * y * y)
    return y


@functools.partial(
    pl.kernel,
    mesh=plsc.VectorSubcoreMesh(core_axis_name="c", subcore_axis_name="s"),
    out_type=jax.ShapeDtypeStruct((N_TOK, DIM), jnp.float32),
    scratch_types=[
        pltpu.VMEM((N_CHUNK, CHUNK), jnp.int32),
        pltpu.VMEM((N_CHUNK, CHUNK), jnp.int32),
        pltpu.VMEM((ROWS_PER_W, DIM), jnp.float32),
        pltpu.VMEM((ROWS_PER_W, DIM), jnp.float32),
        pltpu.VMEM((ROWS_PER_W, DIM), jnp.float32),
        pltpu.VMEM((DIM,), jnp.float32),
        pltpu.VMEM((DIM,), jnp.float32),
        pltpu.SemaphoreType.DMA,
    ],
)
def _embed_ln(word_id_hbm, pos_id_hbm, word_table_hbm, pos_table_hbm,
              gamma_hbm, beta_hbm, out_hbm,
              idx_w, idx_p, wv, pv, ov, gv, bv, sem):
    wid = lax.axis_index("s") * NC + lax.axis_index("c")
    base = wid * ROWS_PER_W
    brow = wid // W_PER_B
    boff = (wid % W_PER_B) * ROWS_PER_W

    for c in range(N_CHUNK):
        src = pl.ds(boff + c * CHUNK, CHUNK)
        pltpu.sync_copy(word_id_hbm.at[brow, src], idx_w.at[c])
        pltpu.sync_copy(pos_id_hbm.at[brow, src], idx_p.at[c])
    pltpu.sync_copy(gamma_hbm, gv)
    pltpu.sync_copy(beta_hbm, bv)

    copies = []
    for c in range(N_CHUNK):
        dst = pl.ds(c * CHUNK, CHUNK)
        copies.append(pltpu.async_copy(
            word_table_hbm.at[idx_w.at[c]], wv.at[dst], sem))
        copies.append(pltpu.async_copy(
            pos_table_hbm.at[idx_p.at[c]], pv.at[dst], sem))
    for cp in copies:
        cp.wait()

    @functools.partial(plsc.parallel_loop, 0, ROWS_PER_W, unroll=1)
    def _(r):
        vals = []
        acc = jnp.zeros((LANES,), jnp.float32)
        acc2 = jnp.zeros((LANES,), jnp.float32)
        for j in range(DLANES):
            sl = pl.ds(j * LANES, LANES)
            v = wv[r, sl] + pv[r, sl]
            vals.append(v)
            acc = acc + v
            acc2 = acc2 + v * v
        s = _xlane_sum(acc)
        s2 = _xlane_sum(acc2)
        mean = s * (1.0 / DIM)
        var = jnp.maximum(s2 * (1.0 / DIM) - mean * mean, 0.0) + EPS
        inv = _rsqrt(var)
        for j in range(DLANES):
            sl = pl.ds(j * LANES, LANES)
            ov[r, sl] = (vals[j] - mean) * inv * gv[sl] + bv[sl]

    pltpu.sync_copy(ov, out_hbm.at[pl.ds(base, ROWS_PER_W)])


def kernel(word_id, position_id, word_table, pos_table, gamma, beta):
    out = _embed_ln(word_id, position_id, word_table, pos_table, gamma, beta)
    return out.reshape(BATCH, SEQ, DIM)


# speedup vs baseline: 1.0137x; 1.0137x over previous
import functools

import jax
import jax.numpy as jnp
from jax import lax
from jax.experimental import pallas as pl
from jax.experimental.pallas import tpu as pltpu
from jax.experimental.pallas import tpu_sc as plsc

VOCAB = 100000
SEQ = 2048
DIM = 128
EPS = 1e-12
BATCH = 4

NC = 2
NS = 16
NW = NC * NS
N_TOK = BATCH * SEQ
ROWS_PER_W = N_TOK // NW
CHUNK = 128
N_CHUNK = ROWS_PER_W // CHUNK
LANES = 16
DLANES = DIM // LANES


def _xlane_sum(x):
    idx = lax.iota(jnp.int32, LANES)
    dnums = lax.GatherDimensionNumbers(
        offset_dims=(), collapsed_slice_dims=(0,), start_index_map=(0,))
    for k in (1, 2, 4, 8):
        perm = lax.bitwise_xor(idx, jnp.int32(k))
        x = x + lax.gather(
            x, perm[:, None], dimension_numbers=dnums, slice_sizes=(1,),
            mode=lax.GatherScatterMode.PROMISE_IN_BOUNDS)
    return x


def _rsqrt(x):
    yi = jnp.int32(0x5F3759DF) - lax.shift_right_logical(
        lax.bitcast_convert_type(x, jnp.int32), 1)
    y = lax.bitcast_convert_type(yi, jnp.float32)
    for _ in range(3):
        y = y * (1.5 - 0.5 * x * y * y)
    return y


@functools.partial(
    pl.kernel,
    mesh=plsc.VectorSubcoreMesh(core_axis_name="c", subcore_axis_name="s"),
    out_type=jax.ShapeDtypeStruct((N_TOK, DIM), jnp.float32),
    scratch_types=[
        pltpu.VMEM((N_CHUNK, CHUNK), jnp.int32),
        pltpu.VMEM((N_CHUNK, CHUNK), jnp.int32),
        pltpu.VMEM((ROWS_PER_W, DIM), jnp.float32),
        pltpu.VMEM((ROWS_PER_W, DIM), jnp.float32),
        pltpu.VMEM((ROWS_PER_W, DIM), jnp.float32),
        pltpu.VMEM((DIM,), jnp.float32),
        pltpu.VMEM((DIM,), jnp.float32),
        pltpu.SemaphoreType.DMA,
    ],
)
def _embed_ln(word_id_hbm, pos_id_hbm, word_table_hbm, pos_table_hbm,
              gamma_hbm, beta_hbm, out_hbm,
              idx_w, idx_p, wv, pv, ov, gv, bv, sem):
    wid = lax.axis_index("s") * NC + lax.axis_index("c")
    base = wid * ROWS_PER_W
    pltpu.sync_copy(word_id_hbm.at[wid], idx_w)
    pltpu.sync_copy(pos_id_hbm.at[wid], idx_p)
    pltpu.sync_copy(gamma_hbm, gv)
    pltpu.sync_copy(beta_hbm, bv)

    copies = []
    for c in range(N_CHUNK):
        dst = pl.ds(c * CHUNK, CHUNK)
        copies.append(pltpu.async_copy(
            word_table_hbm.at[idx_w.at[c]], wv.at[dst], sem))
        copies.append(pltpu.async_copy(
            pos_table_hbm.at[idx_p.at[c]], pv.at[dst], sem))
    for cp in copies:
        cp.wait()

    @functools.partial(plsc.parallel_loop, 0, ROWS_PER_W, unroll=1)
    def _(r):
        vals = []
        acc = jnp.zeros((LANES,), jnp.float32)
        acc2 = jnp.zeros((LANES,), jnp.float32)
        for j in range(DLANES):
            sl = pl.ds(j * LANES, LANES)
            v = wv[r, sl] + pv[r, sl]
            vals.append(v)
            acc = acc + v
            acc2 = acc2 + v * v
        s = _xlane_sum(acc)
        s2 = _xlane_sum(acc2)
        mean = s * (1.0 / DIM)
        var = jnp.maximum(s2 * (1.0 / DIM) - mean * mean, 0.0) + EPS
        inv = _rsqrt(var)
        for j in range(DLANES):
            sl = pl.ds(j * LANES, LANES)
            ov[r, sl] = (vals[j] - mean) * inv * gv[sl] + bv[sl]

    pltpu.sync_copy(ov, out_hbm.at[pl.ds(base, ROWS_PER_W)])


def kernel(word_id, position_id, word_table, pos_table, gamma, beta):
    wid = word_id.astype(jnp.int32).reshape(NW, N_CHUNK, CHUNK)
    pid = position_id.astype(jnp.int32).reshape(NW, N_CHUNK, CHUNK)
    out = _embed_ln(wid, pid, word_table, pos_table, gamma, beta)
    return out.reshape(BATCH, SEQ, DIM)
